# baseline (speedup 1.0000x reference)
import functools
import jax
import jax.numpy as jnp
from jax import lax
from jax.experimental import pallas as pl
from jax.experimental.pallas import tpu as pltpu
from jax.experimental.pallas import tpu_sc as plsc

_H, _W, _NH = 32, 32, 16
_L = _H * _W
_NC = 2
_KPAD = 64


def _sc_body(rh_hbm, rw_hbm, out_hbm, u_v, v_v, abuf, buf0, buf1, sem0, sem1):
    wid = lax.axis_index("s") * _NC + lax.axis_index("c")
    h = wid // 2
    half = wid % 2

    pltpu.sync_copy(rh_hbm.at[h], u_v)
    pltpu.sync_copy(rw_hbm.at[h], v_v)

    lane = lax.broadcasted_iota(jnp.int32, (16,), 0)
    zeros16 = jnp.zeros((16,), jnp.int32)

    def fill_chunk(ri, buf, sem):
        for g in range(_H):
            abuf[g, :] = plsc.load_gather(u_v, [zeros16 + (ri + 31 - g)])

        def ci_body(ci, carry):
            b_lo = plsc.load_gather(v_v, [(ci + 31) - lane])
            b_hi = plsc.load_gather(v_v, [(ci + 15) - lane])
            for g in range(_H):
                a = abuf[g, :]
                buf[ci, pl.ds(32 * g, 16)] = b_lo + a
                buf[ci, pl.ds(32 * g + 16, 16)] = b_hi + a
            return carry

        lax.fori_loop(0, _W, ci_body, 0)
        pltpu.async_copy(buf, out_hbm.at[h, pl.ds(ri * 32, 32)], sem)

    def body(k, carry):
        ri0 = half * 16 + 2 * k

        @pl.when(k > 0)
        def _():
            pltpu.make_async_copy(
                buf0, out_hbm.at[h, pl.ds((ri0 - 2) * 32, 32)], sem0
            ).wait()

        fill_chunk(ri0, buf0, sem0)

        @pl.when(k > 0)
        def _():
            pltpu.make_async_copy(
                buf1, out_hbm.at[h, pl.ds((ri0 - 1) * 32, 32)], sem1
            ).wait()

        fill_chunk(ri0 + 1, buf1, sem1)
        return carry

    lax.fori_loop(0, 8, body, 0)

    last0 = half * 16 + 14
    pltpu.make_async_copy(buf0, out_hbm.at[h, pl.ds(last0 * 32, 32)], sem0).wait()
    pltpu.make_async_copy(
        buf1, out_hbm.at[h, pl.ds((last0 + 1) * 32, 32)], sem1
    ).wait()


def kernel(rel_height, rel_width):
    rh = jnp.zeros((_NH, _KPAD), jnp.float32).at[:, : 2 * _H - 1].set(rel_height.T)
    rw = jnp.zeros((_NH, _KPAD), jnp.float32).at[:, : 2 * _W - 1].set(rel_width.T)
    mesh = plsc.VectorSubcoreMesh(core_axis_name="c", subcore_axis_name="s")
    f = pl.kernel(
        _sc_body,
        out_type=jax.ShapeDtypeStruct((_NH, _L, _L), jnp.float32),
        mesh=mesh,
        scratch_types=[
            pltpu.VMEM((_KPAD,), jnp.float32),
            pltpu.VMEM((_KPAD,), jnp.float32),
            pltpu.VMEM((_H, 16), jnp.float32),
            pltpu.VMEM((_H, _L), jnp.float32),
            pltpu.VMEM((_H, _L), jnp.float32),
            pltpu.SemaphoreType.DMA,
            pltpu.SemaphoreType.DMA,
        ],
        compiler_params=pltpu.CompilerParams(needs_layout_passes=False),
    )
    return f(rh, rw)


# speedup vs baseline: 2.1504x; 2.1504x over previous
import functools
import jax
import jax.numpy as jnp
from jax import lax
from jax.experimental import pallas as pl
from jax.experimental.pallas import tpu as pltpu
from jax.experimental.pallas import tpu_sc as plsc

_H, _W, _NH = 32, 32, 16
_L = _H * _W
_NC = 2
_KPAD = 64


def _sc_body(rh_hbm, rw_hbm, out_hbm, u_v, v_v, abuf, buf0, buf1, sem0, sem1):
    wid = lax.axis_index("s") * _NC + lax.axis_index("c")
    h = wid // 2
    half = wid % 2

    pltpu.sync_copy(rh_hbm.at[h], u_v)
    pltpu.sync_copy(rw_hbm.at[h], v_v)

    lane = lax.broadcasted_iota(jnp.int32, (16,), 0)
    zeros16 = jnp.zeros((16,), jnp.int32)

    def fill_chunk(ri, buf, sem):
        for g in range(_H):
            abuf[g, :] = plsc.load_gather(u_v, [zeros16 + (ri + 31 - g)])

        @plsc.parallel_loop(0, _W, unroll=2)
        def ci_body(ci):
            b_lo = plsc.load_gather(v_v, [(ci + 31) - lane])
            b_hi = plsc.load_gather(v_v, [(ci + 15) - lane])
            for g in range(_H):
                a = abuf[g, :]
                buf[ci, pl.ds(32 * g, 16)] = b_lo + a
                buf[ci, pl.ds(32 * g + 16, 16)] = b_hi + a
        pltpu.async_copy(buf, out_hbm.at[h, pl.ds(ri * 32, 32)], sem)

    def body(k, carry):
        ri0 = half * 16 + 2 * k

        @pl.when(k > 0)
        def _():
            pltpu.make_async_copy(
                buf0, out_hbm.at[h, pl.ds((ri0 - 2) * 32, 32)], sem0
            ).wait()

        fill_chunk(ri0, buf0, sem0)

        @pl.when(k > 0)
        def _():
            pltpu.make_async_copy(
                buf1, out_hbm.at[h, pl.ds((ri0 - 1) * 32, 32)], sem1
            ).wait()

        fill_chunk(ri0 + 1, buf1, sem1)
        return carry

    lax.fori_loop(0, 8, body, 0)

    last0 = half * 16 + 14
    pltpu.make_async_copy(buf0, out_hbm.at[h, pl.ds(last0 * 32, 32)], sem0).wait()
    pltpu.make_async_copy(
        buf1, out_hbm.at[h, pl.ds((last0 + 1) * 32, 32)], sem1
    ).wait()


def kernel(rel_height, rel_width):
    rh = jnp.zeros((_NH, _KPAD), jnp.float32).at[:, : 2 * _H - 1].set(rel_height.T)
    rw = jnp.zeros((_NH, _KPAD), jnp.float32).at[:, : 2 * _W - 1].set(rel_width.T)
    mesh = plsc.VectorSubcoreMesh(core_axis_name="c", subcore_axis_name="s")
    f = pl.kernel(
        _sc_body,
        out_type=jax.ShapeDtypeStruct((_NH, _L, _L), jnp.float32),
        mesh=mesh,
        scratch_types=[
            pltpu.VMEM((_KPAD,), jnp.float32),
            pltpu.VMEM((_KPAD,), jnp.float32),
            pltpu.VMEM((_H, 16), jnp.float32),
            pltpu.VMEM((_H, _L), jnp.float32),
            pltpu.VMEM((_H, _L), jnp.float32),
            pltpu.SemaphoreType.DMA,
            pltpu.SemaphoreType.DMA,
        ],
        compiler_params=pltpu.CompilerParams(needs_layout_passes=False),
    )
    return f(rh, rw)
